# baseline (speedup 1.0000x reference)
import jax
import jax.numpy as jnp
from jax import lax
from jax.experimental import pallas as pl
from jax.experimental.pallas import tpu as pltpu

_EPS_BN = 1e-5
_EPS_LN = 1e-5
_TEMP = 2.0
_ENT_LAMBDA = 0.1
_F32 = jnp.float32


def _cparams(**kw):
    return pltpu.CompilerParams(vmem_limit_bytes=48 * 1024 * 1024, **kw)


def _t0_body(x_ref, tm_ref, hsq_ref):
    x = x_ref[...]
    tm_ref[...] = jnp.sum(jnp.sum(x, axis=3), axis=2) * (1.0 / 1024.0)
    hsq_ref[...] = jnp.sum(x * x, axis=1) * (1.0 / 512.0)


def _t1_body(x_ref, tm_ref, hsq_ref):
    x = x_ref[...]
    y = jnp.sum(x, axis=3).reshape(x.shape[0], 256, 4, 2)
    tm_ref[...] = jnp.sum(y, axis=3) * (1.0 / 256.0)
    z = jnp.sum(x * x, axis=1).reshape(x.shape[0], 4, 2, 128)
    hsq_ref[...] = jnp.sum(z, axis=1) * (1.0 / 1024.0)


def _t2_body(x_ref, tme_ref, tmo_ref, hsq_ref):
    x = x_ref[...]
    tme_ref[...] = jnp.sum(x[..., 0:64], axis=3) * (1.0 / 64.0)
    tmo_ref[...] = jnp.sum(x[..., 64:128], axis=3) * (1.0 / 64.0)
    z = jnp.sum(x * x, axis=1)
    z = jnp.sum(z, axis=1)
    hsq_ref[...] = (z[:, 0:64] + z[:, 64:128]) * (1.0 / 2048.0)


def _t3_body(x_ref, tm_ref, hsq_ref):
    x = x_ref[...]
    tm_ref[...] = jnp.mean(x, axis=2)
    hsq_ref[...] = jnp.mean(x * x, axis=1)


def _teacher_call(body, x, bb, out_shapes):
    bs = x.shape[0]
    rest = x.shape[1:]
    outs = []
    specs = []
    for s in out_shapes:
        outs.append(jax.ShapeDtypeStruct((bs,) + s, _F32))
        specs.append(pl.BlockSpec((bb,) + s,
                                  lambda i, n=len(s): (i,) + (0,) * n))
    return pl.pallas_call(
        body,
        grid=(bs // bb,),
        in_specs=[pl.BlockSpec((bb,) + rest,
                               lambda i, n=len(rest): (i,) + (0,) * n)],
        out_specs=specs,
        out_shape=outs,
        compiler_params=_cparams(dimension_semantics=("parallel",)),
        name=body.__name__.strip("_"),
    )(x)


def _student_body(*refs):
    xs = refs[:16]
    cm_ref, p_ref = refs[16], refs[17]
    for s in range(16):
        x = xs[s][...]
        sm = jnp.sum(jnp.sum(x, axis=3), axis=2) * (1.0 / 1024.0)
        mx = jnp.max(jnp.max(x, axis=3), axis=2)
        cm_ref[:, s, :] = 0.7 * sm + 0.3 * mx
        p_ref[:, s, :, :] = jnp.sum(x * x, axis=1) * (1.0 / 16.0)


def _student_reduce(g_s, bb):
    bs = g_s[0].shape[0]
    xs = [g.reshape(bs, 16, 8, 128) for g in g_s]
    spec = pl.BlockSpec((bb, 16, 8, 128), lambda i: (i, 0, 0, 0))
    return pl.pallas_call(
        _student_body,
        grid=(bs // bb,),
        in_specs=[spec] * 16,
        out_specs=[pl.BlockSpec((bb, 16, 16), lambda i: (i, 0, 0)),
                   pl.BlockSpec((bb, 16, 8, 128), lambda i: (i, 0, 0, 0))],
        out_shape=[jax.ShapeDtypeStruct((bs, 16, 16), _F32),
                   jax.ShapeDtypeStruct((bs, 16, 8, 128), _F32)],
        compiler_params=_cparams(dimension_semantics=("parallel",)),
        name="student_reduce",
    )(*xs)


def _bn_batch(x, g, b):
    mu = jnp.mean(x, axis=0, keepdims=True)
    xc = x - mu
    v = jnp.mean(xc * xc, axis=0, keepdims=True)
    return xc * lax.rsqrt(v + _EPS_BN) * g + b


def _l2n(x):
    n = jnp.sqrt(jnp.sum(x * x, axis=1, keepdims=True))
    return x / jnp.maximum(n, 1e-12)


def _sig(x):
    return 1.0 / (1.0 + jnp.exp(-x))


def _epilogue_body(cm_ref, p_ref, q2_ref, q4_ref, q8_ref,
                   tm0_ref, tm1_ref, tm2e_ref, tm2o_ref, tm3_ref,
                   hs0_ref, hs1_ref, hs2_ref, hs3_ref,
                   aw_ref, ab_ref,
                   kW_ref, kb_ref, kg_ref, kbeta_ref,
                   W1_ref, b1_ref, g1_ref, beta1_ref,
                   W2_ref, b2_ref, g2_ref, beta2_ref,
                   qW0_ref, qW1_ref, qW2e_ref, qW2o_ref, qW3_ref,
                   qb_ref, qg_ref, qbeta_ref,
                   pt_ref, ps_ref, lw_ref,
                   lng0_ref, lng1_ref, lng2_ref, lng3_ref,
                   lnb0_ref, lnb1_ref, lnb2_ref, lnb3_ref,
                   out_ref):
    cn = (((1,), (1,)), ((), ()))

    cm = cm_ref[...]
    ks = []
    for s in range(16):
        k_s = lax.dot_general(cm[:, s, :], kW_ref[s], cn,
                              preferred_element_type=_F32) + kb_ref[s]
        k_s = jnp.maximum(_bn_batch(k_s, kg_ref[s], kbeta_ref[s]), 0.0)
        ks.append(k_s[:, None, :])
    key2 = jnp.concatenate(ks, axis=1).reshape(1024, 128)

    h1 = lax.dot_general(key2, W1_ref[...], cn,
                         preferred_element_type=_F32) + b1_ref[...]
    h1 = jnp.maximum(_bn_batch(h1, g1_ref[...], beta1_ref[...]), 0.0)
    h2 = lax.dot_general(h1, W2_ref[...], cn,
                         preferred_element_type=_F32) + b2_ref[...]
    h2 = jnp.maximum(_bn_batch(h2, g2_ref[...], beta2_ref[...]), 0.0)

    q2 = (lax.dot_general(tm2e_ref[...], qW2e_ref[...], cn,
                          preferred_element_type=_F32)
          + lax.dot_general(tm2o_ref[...], qW2o_ref[...], cn,
                            preferred_element_type=_F32))
    qs = [lax.dot_general(tm0_ref[...], qW0_ref[...], cn,
                          preferred_element_type=_F32),
          lax.dot_general(tm1_ref[...], qW1_ref[...], cn,
                          preferred_element_type=_F32),
          q2,
          lax.dot_general(tm3_ref[...], qW3_ref[...], cn,
                          preferred_element_type=_F32)]
    nqs = []
    for t in range(4):
        q = _bn_batch(qs[t] + qb_ref[t], qg_ref[t], qbeta_ref[t])
        nqs.append(_l2n(q))

    pp = lax.dot_general(pt_ref[...], ps_ref[...], cn,
                         preferred_element_type=_F32)
    atts = []
    ent_acc = jnp.zeros((1, 1), _F32)
    for t in range(4):
        nk = _l2n(h2[:, t * 128:(t + 1) * 128])
        cos = jnp.sum(nk.reshape(64, 16, 128) * nqs[t][:, None, :], axis=2)
        logit = (cos + pp[t]) * (1.0 / _TEMP)
        m = jnp.max(logit, axis=1, keepdims=True)
        e = jnp.exp(logit - m)
        att = e / jnp.sum(e, axis=1, keepdims=True)
        atts.append(att)
        ent_acc = ent_acc + jnp.sum(att * jnp.log(att + 1e-8), keepdims=True)
    total = _ENT_LAMBDA * (-ent_acc / 256.0)

    lwv = lw_ref[...]
    le = jnp.exp(lwv - jnp.max(lwv, axis=1, keepdims=True))
    wts = le / jnp.sum(le, axis=1, keepdims=True)

    p3 = p_ref[...]
    pm = jnp.mean(p3, axis=2, keepdims=True)
    sigs = [_sig(aw_ref[0, t] * pm + ab_ref[0, t]) for t in range(4)]

    m2 = 0.25 * (q2_ref[0] + q2_ref[1] + q2_ref[2] + q2_ref[3])
    x2 = jnp.maximum(jnp.maximum(q2_ref[0], q2_ref[1]),
                     jnp.maximum(q2_ref[2], q2_ref[3]))
    s_acc = q4_ref[0]
    x_acc = q4_ref[0]
    for j in range(1, 16):
        xj = q4_ref[j]
        s_acc = s_acc + xj
        x_acc = jnp.maximum(x_acc, xj)
    q8 = q8_ref[...]
    m8 = jnp.mean(q8, axis=2)
    x8 = jnp.max(q8, axis=2)

    combs = [sigs[0] * p3,
             sigs[1] * (0.7 * m2 + 0.3 * x2).reshape(64, 16, 256),
             sigs[2] * (0.7 / 16.0 * s_acc + 0.3 * x_acc).reshape(64, 16, 64),
             sigs[3] * (0.7 * m8 + 0.3 * x8).reshape(64, 16, 16)]

    lngs = (lng0_ref, lng1_ref, lng2_ref, lng3_ref)
    lnbs = (lnb0_ref, lnb1_ref, lnb2_ref, lnb3_ref)
    hts = (hs0_ref, hs1_ref, hs2_ref, hs3_ref)
    for t in range(4):
        comb = combs[t]
        hw = comb.shape[2]
        nrm = jnp.sqrt(jnp.sum(comb * comb, axis=2, keepdims=True))
        v3 = comb / jnp.maximum(nrm, 1e-12)
        mu = jnp.mean(v3, axis=1, keepdims=True)
        xc = v3 - mu
        var = jnp.mean(xc * xc, axis=1, keepdims=True)
        hn = xc * lax.rsqrt(var + _EPS_LN) * lngs[t][...][None] + lnbs[t][...][None]
        ht = hts[t][...]
        htn = ht / jnp.maximum(jnp.sqrt(jnp.sum(ht * ht, axis=1, keepdims=True)),
                               1e-12)
        d = hn - htn[:, None, :]
        ad = jnp.abs(d)
        sm = jnp.mean(jnp.where(ad < 1.0, 0.5 * d * d, ad - 0.5), axis=2)
        mse = jnp.mean(d * d, axis=2)
        diff = 0.7 * sm + 0.3 * mse * mse
        total = total + (jnp.sum(diff * atts[t], keepdims=True) / 64.0) * wts[:, t:t + 1]

    out_ref[...] = total


def _epilogue(cm, p, q2, q4, q8, tms, hsqs, params):
    n_in = 5 + len(tms) + len(hsqs)
    in_specs = ([pl.BlockSpec(memory_space=pltpu.VMEM)] * n_in
                + [pl.BlockSpec(memory_space=pltpu.SMEM)] * 2
                + [pl.BlockSpec(memory_space=pltpu.VMEM)] * (len(params) - 2))
    return pl.pallas_call(
        _epilogue_body,
        in_specs=in_specs,
        out_specs=pl.BlockSpec(memory_space=pltpu.VMEM),
        out_shape=jax.ShapeDtypeStruct((1, 1), _F32),
        compiler_params=_cparams(),
        name="afd_epilogue",
    )(cm, p, q2, q4, q8, *tms, *hsqs, *params)


def kernel(g_s_0, g_s_1, g_s_2, g_s_3, g_s_4, g_s_5, g_s_6, g_s_7,
           g_s_8, g_s_9, g_s_10, g_s_11, g_s_12, g_s_13, g_s_14, g_s_15,
           g_t_0, g_t_1, g_t_2, g_t_3,
           attn_w, attn_b, key_W, key_b, key_g, key_beta,
           W1, b1, g1, beta1, W2, b2, g2, beta2,
           q_W0, q_W1, q_W2, q_W3, q_b, q_g, q_beta,
           p_t, p_s, layer_weights, ln_g, ln_b):
    g_s = [g_s_0, g_s_1, g_s_2, g_s_3, g_s_4, g_s_5, g_s_6, g_s_7,
           g_s_8, g_s_9, g_s_10, g_s_11, g_s_12, g_s_13, g_s_14, g_s_15]

    tm0, hsq0 = _teacher_call(_t0_body, g_t_0.reshape(64, 512, 8, 128), 8,
                              [(512,), (8, 128)])
    tm1, hsq1 = _teacher_call(_t1_body, g_t_1.reshape(64, 256, 8, 128), 8,
                              [(256, 4), (2, 128)])
    tm2e, tm2o, hsq2 = _teacher_call(_t2_body, g_t_2.reshape(64, 128, 8, 128), 8,
                                     [(128, 8), (128, 8), (64,)])
    tm3, hsq3 = _teacher_call(_t3_body, g_t_3.reshape(64, 2048, 16), 8,
                              [(2048,), (16,)])

    cm, p = _student_reduce(g_s, bb=8)
    p2 = p.reshape(64, 16, 1024)

    q2 = p2.reshape(64, 16, 16, 2, 16, 2).transpose(3, 5, 0, 1, 2, 4) \
           .reshape(4, 1024, 256)
    q4 = p2.reshape(64, 16, 8, 4, 8, 4).transpose(3, 5, 0, 1, 2, 4) \
           .reshape(16, 1024, 64)
    q8 = p2.reshape(64, 16, 4, 8, 4, 8).transpose(0, 1, 2, 4, 3, 5) \
           .reshape(1024, 16, 64)

    tms = [tm0, tm1.reshape(64, 1024), tm2e.reshape(64, 1024),
           tm2o.reshape(64, 1024), tm3]
    hsqs = [hsq0.reshape(64, 1024), hsq1.reshape(64, 256), hsq2, hsq3]

    hws = (1024, 256, 64, 16)
    params = [attn_w.reshape(1, 4), attn_b.reshape(1, 4),
              key_W, key_b, key_g, key_beta,
              W1, b1.reshape(1, 256), g1.reshape(1, 256), beta1.reshape(1, 256),
              W2, b2.reshape(1, 512), g2.reshape(1, 512), beta2.reshape(1, 512),
              q_W0, q_W1, q_W2[:, 0::2], q_W2[:, 1::2], q_W3,
              q_b, q_g, q_beta,
              p_t, p_s, layer_weights.reshape(1, 4)]
    params += [jnp.broadcast_to(ln_g[t, :, None], (16, hws[t])) for t in range(4)]
    params += [jnp.broadcast_to(ln_b[t, :, None], (16, hws[t])) for t in range(4)]

    out = _epilogue(cm, p2, q2, q4, q8, tms, hsqs, params)
    return out.reshape(())
